# baseline (device time: 202462 ns/iter reference)
import jax
import jax.numpy as jnp
from jax import lax
from jax.experimental import pallas as pl
from jax.experimental.pallas import tpu as pltpu

M, N = 16384, 1024
Q = M // 4
PIECES = [256, 256, 512, 512, 512, 512, 512, 512, 256, 256]
NP = len(PIECES)
OFFS = [sum(PIECES[:i]) for i in range(NP)]
MAXP = max(PIECES)
assert sum(PIECES) == Q

_HBM = pltpu.MemorySpace.HBM
_MESH = pl.DeviceIdType.MESH


def kernel(x):
    def body(x_hbm, out_hbm, sq, rz, rx, ry, rd, xv0, xv1, ov0, ov1,
             lsems, osems, sz, rzs, sx1, rx1, sy1, ry1, sx2, rx2,
             sy2, ry2):
        my_x = lax.axis_index("x")
        my_y = lax.axis_index("y")
        my_z = lax.axis_index("z")
        zp = (my_x, my_y, 1 - my_z)
        xn = (1 - my_x, my_y, my_z)
        yn = (my_x, 1 - my_y, my_z)
        q = 2 * my_x + my_y
        qx = 2 * (1 - my_x) + my_y
        qy = 2 * my_x + (1 - my_y)
        qd = 2 * (1 - my_x) + (1 - my_y)
        xvs = [xv0, xv1]
        ovs = [ov0, ov1]

        def piece(buf, p):
            return buf.at[pl.ds(OFFS[p], PIECES[p]), :]

        def in_dma(p):
            cp = pltpu.make_async_copy(
                x_hbm.at[pl.ds(q * Q + OFFS[p], PIECES[p]), :],
                xvs[p % 2].at[pl.ds(0, PIECES[p]), :],
                lsems.at[p % 2])
            cp.start()
            return cp

        in_pending = in_dma(0)

        barrier = pltpu.get_barrier_semaphore()
        for nbr in (zp, xn, yn):
            pl.semaphore_signal(barrier, inc=1, device_id=nbr,
                                device_id_type=_MESH)
        pl.semaphore_wait(barrier, 3)

        out_pending = [None, None]
        store_ct = [0]

        def acquire():
            i = store_ct[0] % 2
            store_ct[0] += 1
            if out_pending[i] is not None:
                out_pending[i].wait()
            return i

        def commit(i, quarter, p):
            cp = pltpu.make_async_copy(
                ovs[i].at[pl.ds(0, PIECES[p]), :],
                out_hbm.at[pl.ds(quarter * Q + OFFS[p], PIECES[p]), :],
                osems.at[i])
            cp.start()
            out_pending[i] = cp

        def store_piece(src, p, quarter):
            i = acquire()
            ovs[i][pl.ds(0, PIECES[p]), :] = (
                src[pl.ds(OFFS[p], PIECES[p]), :].astype(jnp.float32))
            commit(i, quarter, p)

        z_rdmas, r1x_rdmas, r1y_rdmas, r2_rdmas = [], [], [], []
        for p in range(NP):
            in_pending.wait()
            if p + 1 < NP:
                in_pending = in_dma(p + 1)
            sq[pl.ds(OFFS[p], PIECES[p]), :] = (
                xvs[p % 2][pl.ds(0, PIECES[p]), :].astype(jnp.bfloat16))
            rdma = pltpu.make_async_remote_copy(
                src_ref=piece(sq, p), dst_ref=piece(rz, p),
                send_sem=sz.at[p], recv_sem=rzs.at[p],
                device_id=zp, device_id_type=_MESH)
            rdma.start()
            z_rdmas.append(rdma)

        def process_z(p):
            z_rdmas[p].wait_send()
            z_rdmas[p].wait_recv()
            sl = pl.ds(OFFS[p], PIECES[p])
            vsl = pl.ds(0, PIECES[p])
            i = acquire()
            ovs[i][vsl, :] = (sq[sl, :].astype(jnp.float32)
                              + rz[sl, :].astype(jnp.float32))
            sq[sl, :] = ovs[i][vsl, :].astype(jnp.bfloat16)
            r1x = pltpu.make_async_remote_copy(
                src_ref=piece(sq, p), dst_ref=piece(rx, p),
                send_sem=sx1.at[p], recv_sem=rx1.at[p],
                device_id=xn, device_id_type=_MESH)
            r1y = pltpu.make_async_remote_copy(
                src_ref=piece(sq, p), dst_ref=piece(ry, p),
                send_sem=sy1.at[p], recv_sem=ry1.at[p],
                device_id=yn, device_id_type=_MESH)
            if p % 2 == 0:
                r1x.start()
                r1y.start()
            else:
                r1y.start()
                r1x.start()
            r1x_rdmas.append(r1x)
            r1y_rdmas.append(r1y)
            commit(i, q, p)

        def process_r1(p):
            if p % 2 == 0:
                r1y_rdmas[p].wait_recv()
                rdma = pltpu.make_async_remote_copy(
                    src_ref=piece(ry, p), dst_ref=piece(rd, p),
                    send_sem=sx2.at[p], recv_sem=rx2.at[p],
                    device_id=xn, device_id_type=_MESH)
                rdma.start()
                r2_rdmas.append(rdma)
                r1x_rdmas[p].wait_recv()
            else:
                r1x_rdmas[p].wait_recv()
                rdma = pltpu.make_async_remote_copy(
                    src_ref=piece(rx, p), dst_ref=piece(rd, p),
                    send_sem=sy2.at[p], recv_sem=ry2.at[p],
                    device_id=yn, device_id_type=_MESH)
                rdma.start()
                r2_rdmas.append(rdma)
                r1y_rdmas[p].wait_recv()
            store_piece(ry, p, qy)
            store_piece(rx, p, qx)

        for p in range(NP):
            process_z(p)
            if p >= 1:
                process_r1(p - 1)
        process_r1(NP - 1)

        for p in range(NP):
            rdma = pltpu.make_async_remote_copy(
                src_ref=piece(ry, p), dst_ref=piece(rd, p),
                send_sem=sx2.at[p] if p % 2 == 0 else sy2.at[p],
                recv_sem=rx2.at[p] if p % 2 == 0 else ry2.at[p],
                device_id=xn if p % 2 == 0 else yn,
                device_id_type=_MESH)
            rdma.wait_recv()
            store_piece(rd, p, qd)

        for rdma in r1x_rdmas + r1y_rdmas + r2_rdmas:
            rdma.wait_send()
        for cp in out_pending:
            if cp is not None:
                cp.wait()

    return pl.pallas_call(
        body,
        out_shape=jax.ShapeDtypeStruct((M, N), jnp.float32),
        in_specs=[pl.BlockSpec(memory_space=_HBM)],
        out_specs=pl.BlockSpec(memory_space=_HBM),
        scratch_shapes=[
            pltpu.VMEM((Q, N), jnp.bfloat16),
            pltpu.VMEM((Q, N), jnp.bfloat16),
            pltpu.VMEM((Q, N), jnp.bfloat16),
            pltpu.VMEM((Q, N), jnp.bfloat16),
            pltpu.VMEM((Q, N), jnp.bfloat16),
            pltpu.VMEM((MAXP, N), jnp.float32),
            pltpu.VMEM((MAXP, N), jnp.float32),
            pltpu.VMEM((MAXP, N), jnp.float32),
            pltpu.VMEM((MAXP, N), jnp.float32),
            pltpu.SemaphoreType.DMA((2,)),
            pltpu.SemaphoreType.DMA((2,)),
            pltpu.SemaphoreType.DMA((NP,)),
            pltpu.SemaphoreType.DMA((NP,)),
            pltpu.SemaphoreType.DMA((NP,)),
            pltpu.SemaphoreType.DMA((NP,)),
            pltpu.SemaphoreType.DMA((NP,)),
            pltpu.SemaphoreType.DMA((NP,)),
            pltpu.SemaphoreType.DMA((NP,)),
            pltpu.SemaphoreType.DMA((NP,)),
            pltpu.SemaphoreType.DMA((NP,)),
            pltpu.SemaphoreType.DMA((NP,)),
        ],
        compiler_params=pltpu.CompilerParams(
            collective_id=0, vmem_limit_bytes=56 * 1024 * 1024
        ),
    )(x)


# device time: 190934 ns/iter; 1.0604x vs baseline; 1.0604x over previous
import jax
import jax.numpy as jnp
from jax import lax
from jax.experimental import pallas as pl
from jax.experimental.pallas import tpu as pltpu

M, N = 16384, 1024
Q = M // 4
PIECES = [256, 256, 512, 512, 512, 512, 512, 512, 256, 256]
NP = len(PIECES)
OFFS = [sum(PIECES[:i]) for i in range(NP)]
MAXP = max(PIECES)
assert sum(PIECES) == Q

A_SET = {0, 1, 8, 9}
VIA_X = {0, 8, 3, 7, 2, 6}

_HBM = pltpu.MemorySpace.HBM
_MESH = pl.DeviceIdType.MESH


def kernel(x):
    def body(x_hbm, out_hbm, sq, rz, rx, ry, rd, xv0, xv1, ov0, ov1,
             lsems, osems, sz, rzs, sx1, rx1, sy1, ry1, s2, rdsems,
             s2z):
        my_x = lax.axis_index("x")
        my_y = lax.axis_index("y")
        my_z = lax.axis_index("z")
        zp = (my_x, my_y, 1 - my_z)
        xn = (1 - my_x, my_y, my_z)
        yn = (my_x, 1 - my_y, my_z)
        q = 2 * my_x + my_y
        qx = 2 * (1 - my_x) + my_y
        qy = 2 * my_x + (1 - my_y)
        qd = 2 * (1 - my_x) + (1 - my_y)
        xvs = [xv0, xv1]
        ovs = [ov0, ov1]

        def owner_cond(p):
            return (my_z == 0) if p % 2 == 1 else (my_z == 1)

        def piece(buf, p):
            return buf.at[pl.ds(OFFS[p], PIECES[p]), :]

        def in_dma(p):
            cp = pltpu.make_async_copy(
                x_hbm.at[pl.ds(q * Q + OFFS[p], PIECES[p]), :],
                xvs[p % 2].at[pl.ds(0, PIECES[p]), :],
                lsems.at[p % 2])
            cp.start()
            return cp

        in_pending = in_dma(0)

        barrier = pltpu.get_barrier_semaphore()
        for nbr in (zp, xn, yn):
            pl.semaphore_signal(barrier, inc=1, device_id=nbr,
                                device_id_type=_MESH)
        pl.semaphore_wait(barrier, 3)

        out_pending = [None, None]
        store_ct = [0]

        def acquire():
            i = store_ct[0] % 2
            store_ct[0] += 1
            if out_pending[i] is not None:
                out_pending[i].wait()
            return i

        def commit(i, quarter, p):
            cp = pltpu.make_async_copy(
                ovs[i].at[pl.ds(0, PIECES[p]), :],
                out_hbm.at[pl.ds(quarter * Q + OFFS[p], PIECES[p]), :],
                osems.at[i])
            cp.start()
            out_pending[i] = cp

        def store_piece(src, p, quarter):
            i = acquire()
            ovs[i][pl.ds(0, PIECES[p]), :] = (
                src[pl.ds(OFFS[p], PIECES[p]), :].astype(jnp.float32))
            commit(i, quarter, p)

        z_rdmas, r1x_rdmas, r1y_rdmas = [], [], []
        cond_sends = []
        for p in range(NP):
            in_pending.wait()
            if p + 1 < NP:
                in_pending = in_dma(p + 1)
            sq[pl.ds(OFFS[p], PIECES[p]), :] = (
                xvs[p % 2][pl.ds(0, PIECES[p]), :].astype(jnp.bfloat16))
            rdma = pltpu.make_async_remote_copy(
                src_ref=piece(sq, p), dst_ref=piece(rz, p),
                send_sem=sz.at[p], recv_sem=rzs.at[p],
                device_id=zp, device_id_type=_MESH)
            rdma.start()
            z_rdmas.append(rdma)

        def process_z(p):
            z_rdmas[p].wait_send()
            z_rdmas[p].wait_recv()
            sl = pl.ds(OFFS[p], PIECES[p])
            vsl = pl.ds(0, PIECES[p])
            i = acquire()
            ovs[i][vsl, :] = (sq[sl, :].astype(jnp.float32)
                              + rz[sl, :].astype(jnp.float32))
            sq[sl, :] = ovs[i][vsl, :].astype(jnp.bfloat16)
            r1x = pltpu.make_async_remote_copy(
                src_ref=piece(sq, p), dst_ref=piece(rx, p),
                send_sem=sx1.at[p], recv_sem=rx1.at[p],
                device_id=xn, device_id_type=_MESH)
            r1y = pltpu.make_async_remote_copy(
                src_ref=piece(sq, p), dst_ref=piece(ry, p),
                send_sem=sy1.at[p], recv_sem=ry1.at[p],
                device_id=yn, device_id_type=_MESH)
            if p % 2 == 0:
                r1x.start()
                r1y.start()
            else:
                r1y.start()
                r1x.start()
            r1x_rdmas.append(r1x)
            r1y_rdmas.append(r1y)
            commit(i, q, p)

        def process_r1(p):
            if p in VIA_X:
                fwd = pltpu.make_async_remote_copy(
                    src_ref=piece(ry, p), dst_ref=piece(rd, p),
                    send_sem=s2.at[p], recv_sem=rdsems.at[p],
                    device_id=xn, device_id_type=_MESH)
                first, second = r1y_rdmas[p], r1x_rdmas[p]
            else:
                fwd = pltpu.make_async_remote_copy(
                    src_ref=piece(rx, p), dst_ref=piece(rd, p),
                    send_sem=s2.at[p], recv_sem=rdsems.at[p],
                    device_id=yn, device_id_type=_MESH)
                first, second = r1x_rdmas[p], r1y_rdmas[p]
            first.wait_recv()
            if p in A_SET:
                fwd.start()
                cond_sends.append(("always", fwd))
            else:
                @pl.when(owner_cond(p))
                def _():
                    fwd.start()
                cond_sends.append(("odd" if p % 2 == 1 else "even", fwd))
            second.wait_recv()
            store_piece(ry, p, qy)
            store_piece(rx, p, qx)

        def rd_wait_descriptor(p):
            return pltpu.make_async_remote_copy(
                src_ref=piece(rd, p), dst_ref=piece(rd, p),
                send_sem=s2z.at[p], recv_sem=rdsems.at[p],
                device_id=zp, device_id_type=_MESH)

        def process_rd_inplane(p):
            if p in A_SET:
                return
            zfwd = pltpu.make_async_remote_copy(
                src_ref=piece(rd, p), dst_ref=piece(rd, p),
                send_sem=s2z.at[p], recv_sem=rdsems.at[p],
                device_id=zp, device_id_type=_MESH)
            wait_d = rd_wait_descriptor(p)

            @pl.when(owner_cond(p))
            def _():
                wait_d.wait_recv()
                zfwd.start()
            cond_sends.append(("zodd" if p % 2 == 1 else "zeven", zfwd))

        for p in range(NP):
            process_z(p)
            if p >= 1:
                process_r1(p - 1)
            if p >= 2:
                process_rd_inplane(p - 2)
        process_r1(NP - 1)
        process_rd_inplane(NP - 2)
        process_rd_inplane(NP - 1)

        for p in range(NP):
            wait_d = rd_wait_descriptor(p)
            if p in A_SET:
                wait_d.wait_recv()
            else:
                @pl.when(jnp.logical_not(owner_cond(p)))
                def _():
                    wait_d.wait_recv()
            store_piece(rd, p, qd)

        for rdma in r1x_rdmas + r1y_rdmas:
            rdma.wait_send()
        for kind, rdma in cond_sends:
            if kind == "always":
                rdma.wait_send()
            else:
                cond = {
                    "odd": my_z == 0, "even": my_z == 1,
                    "zodd": my_z == 0, "zeven": my_z == 1,
                }[kind]

                @pl.when(cond)
                def _(rdma=rdma):
                    rdma.wait_send()
        for cp in out_pending:
            if cp is not None:
                cp.wait()

    return pl.pallas_call(
        body,
        out_shape=jax.ShapeDtypeStruct((M, N), jnp.float32),
        in_specs=[pl.BlockSpec(memory_space=_HBM)],
        out_specs=pl.BlockSpec(memory_space=_HBM),
        scratch_shapes=[
            pltpu.VMEM((Q, N), jnp.bfloat16),
            pltpu.VMEM((Q, N), jnp.bfloat16),
            pltpu.VMEM((Q, N), jnp.bfloat16),
            pltpu.VMEM((Q, N), jnp.bfloat16),
            pltpu.VMEM((Q, N), jnp.bfloat16),
            pltpu.VMEM((MAXP, N), jnp.float32),
            pltpu.VMEM((MAXP, N), jnp.float32),
            pltpu.VMEM((MAXP, N), jnp.float32),
            pltpu.VMEM((MAXP, N), jnp.float32),
            pltpu.SemaphoreType.DMA((2,)),
            pltpu.SemaphoreType.DMA((2,)),
            pltpu.SemaphoreType.DMA((NP,)),
            pltpu.SemaphoreType.DMA((NP,)),
            pltpu.SemaphoreType.DMA((NP,)),
            pltpu.SemaphoreType.DMA((NP,)),
            pltpu.SemaphoreType.DMA((NP,)),
            pltpu.SemaphoreType.DMA((NP,)),
            pltpu.SemaphoreType.DMA((NP,)),
            pltpu.SemaphoreType.DMA((NP,)),
            pltpu.SemaphoreType.DMA((NP,)),
        ],
        compiler_params=pltpu.CompilerParams(
            collective_id=0, vmem_limit_bytes=56 * 1024 * 1024
        ),
    )(x)
